# baseline (device time: 25165 ns/iter reference)
import jax
import jax.numpy as jnp
from jax import lax
from jax.experimental import pallas as pl
from jax.experimental.pallas import tpu as pltpu

N_DEV = 32
LAST = N_DEV - 1
B = 2
S = 256
H = 128
HQ = 4
DH = 64
W = H + S + H
SQ_GLOBAL = N_DEV * S


def kernel(x, Wq, K_ext, V_ext, Wo):
    def body(x_ref, wq_ref, k_ref, v_ref, wo_ref, out_ref,
             kfull, vfull, stage, send_sems, recv_sems):
        my = lax.axis_index("i")
        left = my - 1
        right = my + 1

        barrier_sem = pltpu.get_barrier_semaphore()

        @pl.when(my > 0)
        def _():
            pl.semaphore_signal(
                barrier_sem, inc=1,
                device_id=(left,), device_id_type=pl.DeviceIdType.MESH,
            )

        @pl.when(my < LAST)
        def _():
            pl.semaphore_signal(
                barrier_sem, inc=1,
                device_id=(right,), device_id_type=pl.DeviceIdType.MESH,
            )

        @pl.when(my > 0)
        def _():
            pl.semaphore_wait(barrier_sem, 1)

        @pl.when(my < LAST)
        def _():
            pl.semaphore_wait(barrier_sem, 1)

        stage[0] = k_ref[:, pl.ds(0, H)].astype(jnp.bfloat16)
        stage[1] = k_ref[:, pl.ds(S - H, H)].astype(jnp.bfloat16)
        stage[2] = v_ref[:, pl.ds(0, H)].astype(jnp.bfloat16)
        stage[3] = v_ref[:, pl.ds(S - H, H)].astype(jnp.bfloat16)

        def rdma_right_k():
            return pltpu.make_async_remote_copy(
                src_ref=stage.at[1], dst_ref=kfull.at[:, pl.ds(0, H)],
                send_sem=send_sems.at[0], recv_sem=recv_sems.at[0],
                device_id=(right,), device_id_type=pl.DeviceIdType.MESH,
            )

        def rdma_right_v():
            return pltpu.make_async_remote_copy(
                src_ref=stage.at[3], dst_ref=vfull.at[:, pl.ds(0, H)],
                send_sem=send_sems.at[2], recv_sem=recv_sems.at[2],
                device_id=(right,), device_id_type=pl.DeviceIdType.MESH,
            )

        def rdma_left_k():
            return pltpu.make_async_remote_copy(
                src_ref=stage.at[0], dst_ref=kfull.at[:, pl.ds(H + S, H)],
                send_sem=send_sems.at[1], recv_sem=recv_sems.at[1],
                device_id=(left,), device_id_type=pl.DeviceIdType.MESH,
            )

        def rdma_left_v():
            return pltpu.make_async_remote_copy(
                src_ref=stage.at[2], dst_ref=vfull.at[:, pl.ds(H + S, H)],
                send_sem=send_sems.at[3], recv_sem=recv_sems.at[3],
                device_id=(left,), device_id_type=pl.DeviceIdType.MESH,
            )

        @pl.when(my < LAST)
        def _():
            rdma_right_k().start()
            rdma_right_v().start()

        @pl.when(my > 0)
        def _():
            rdma_left_k().start()
            rdma_left_v().start()

        @pl.when(my == 0)
        def _():
            kfull[:, pl.ds(0, H)] = jnp.zeros((B, H, HQ, DH), jnp.bfloat16)
            vfull[:, pl.ds(0, H)] = jnp.zeros((B, H, HQ, DH), jnp.bfloat16)

        @pl.when(my == LAST)
        def _():
            kfull[:, pl.ds(H + S, H)] = jnp.zeros((B, H, HQ, DH), jnp.bfloat16)
            vfull[:, pl.ds(H + S, H)] = jnp.zeros((B, H, HQ, DH), jnp.bfloat16)

        kfull[:, pl.ds(H, S)] = k_ref[:, :, :, :].astype(jnp.bfloat16)
        vfull[:, pl.ds(H, S)] = v_ref[:, :, :, :].astype(jnp.bfloat16)

        xv = x_ref[:, :, :].reshape(B * S, 512).astype(jnp.bfloat16)
        q = jnp.dot(xv, wq_ref[:, :].astype(jnp.bfloat16),
                    preferred_element_type=jnp.float32)
        q = q.reshape(B, S, HQ, DH).astype(jnp.bfloat16)

        qi = lax.broadcasted_iota(jnp.int32, (S, W), 0) + my * S
        kj = lax.broadcasted_iota(jnp.int32, (S, W), 1)
        ki = my * S - H + kj
        mask = (jnp.abs(qi - ki) <= H) & (ki >= 0) & (ki < SQ_GLOBAL)

        @pl.when(my > 0)
        def _():
            rdma_right_k().wait_recv()
            rdma_right_v().wait_recv()
            rdma_left_k().wait_send()
            rdma_left_v().wait_send()

        @pl.when(my < LAST)
        def _():
            rdma_left_k().wait_recv()
            rdma_left_v().wait_recv()
            rdma_right_k().wait_send()
            rdma_right_v().wait_send()

        kf = kfull[:, :, :, :]
        vf = vfull[:, :, :, :]
        wo = wo_ref[:, :].astype(jnp.bfloat16)

        for b in range(B):
            ctx_heads = []
            for h in range(HQ):
                qbh = q[b, :, h, :]
                kbh = kf[b, :, h, :]
                vbh = vf[b, :, h, :]
                s = jax.lax.dot_general(
                    qbh, kbh, (((1,), (1,)), ((), ())),
                    preferred_element_type=jnp.float32,
                ) * 0.125
                w = jnp.exp(jnp.where(mask, s, -1e9))
                w = w / jnp.sum(w, axis=-1, keepdims=True)
                ctx_heads.append(jnp.dot(
                    w.astype(jnp.bfloat16), vbh,
                    preferred_element_type=jnp.float32))
            ctx = jnp.concatenate(ctx_heads, axis=-1)
            out_ref[b, :, :] = jnp.dot(
                ctx.astype(jnp.bfloat16), wo,
                preferred_element_type=jnp.float32)

    return pl.pallas_call(
        body,
        out_shape=jax.ShapeDtypeStruct((B, S, 512), jnp.float32),
        in_specs=[pl.BlockSpec(memory_space=pltpu.VMEM)] * 5,
        out_specs=pl.BlockSpec(memory_space=pltpu.VMEM),
        scratch_shapes=[
            pltpu.VMEM((B, W, HQ, DH), jnp.bfloat16),
            pltpu.VMEM((B, W, HQ, DH), jnp.bfloat16),
            pltpu.VMEM((4, B, H, HQ, DH), jnp.bfloat16),
            pltpu.SemaphoreType.DMA((4,)),
            pltpu.SemaphoreType.DMA((4,)),
        ],
        compiler_params=pltpu.CompilerParams(collective_id=0),
    )(x, Wq, K_ext, V_ext, Wo)


# device time: 22511 ns/iter; 1.1179x vs baseline; 1.1179x over previous
import jax
import jax.numpy as jnp
from jax import lax
from jax.experimental import pallas as pl
from jax.experimental.pallas import tpu as pltpu

N_DEV = 32
LAST = N_DEV - 1
B = 2
S = 256
H = 128
HQ = 4
DH = 64
SQ_GLOBAL = N_DEV * S
BF = jnp.bfloat16


def kernel(x, Wq, K_ext, V_ext, Wo):
    def body(x_ref, wq_ref, k_ref, v_ref, wo_ref, out_ref,
             khalo, vhalo, stage, kc, vc, send_sems, recv_sems):
        my = lax.axis_index("i")
        left = my - 1
        right = my + 1

        barrier_sem = pltpu.get_barrier_semaphore()

        @pl.when(my > 0)
        def _():
            pl.semaphore_signal(
                barrier_sem, inc=1,
                device_id=(left,), device_id_type=pl.DeviceIdType.MESH,
            )

        @pl.when(my < LAST)
        def _():
            pl.semaphore_signal(
                barrier_sem, inc=1,
                device_id=(right,), device_id_type=pl.DeviceIdType.MESH,
            )

        @pl.when(my > 0)
        def _():
            pl.semaphore_wait(barrier_sem, 1)

        @pl.when(my < LAST)
        def _():
            pl.semaphore_wait(barrier_sem, 1)

        stage[0] = k_ref[:, pl.ds(0, H)].astype(BF)
        stage[1] = k_ref[:, pl.ds(S - H, H)].astype(BF)
        stage[2] = v_ref[:, pl.ds(0, H)].astype(BF)
        stage[3] = v_ref[:, pl.ds(S - H, H)].astype(BF)

        def rdma_right_k():
            return pltpu.make_async_remote_copy(
                src_ref=stage.at[1], dst_ref=khalo.at[:, pl.ds(0, H)],
                send_sem=send_sems.at[0], recv_sem=recv_sems.at[0],
                device_id=(right,), device_id_type=pl.DeviceIdType.MESH,
            )

        def rdma_right_v():
            return pltpu.make_async_remote_copy(
                src_ref=stage.at[3], dst_ref=vhalo.at[:, pl.ds(0, H)],
                send_sem=send_sems.at[2], recv_sem=recv_sems.at[2],
                device_id=(right,), device_id_type=pl.DeviceIdType.MESH,
            )

        def rdma_left_k():
            return pltpu.make_async_remote_copy(
                src_ref=stage.at[0], dst_ref=khalo.at[:, pl.ds(H, H)],
                send_sem=send_sems.at[1], recv_sem=recv_sems.at[1],
                device_id=(left,), device_id_type=pl.DeviceIdType.MESH,
            )

        def rdma_left_v():
            return pltpu.make_async_remote_copy(
                src_ref=stage.at[2], dst_ref=vhalo.at[:, pl.ds(H, H)],
                send_sem=send_sems.at[3], recv_sem=recv_sems.at[3],
                device_id=(left,), device_id_type=pl.DeviceIdType.MESH,
            )

        @pl.when(my < LAST)
        def _():
            rdma_right_k().start()
            rdma_right_v().start()

        @pl.when(my > 0)
        def _():
            rdma_left_k().start()
            rdma_left_v().start()

        @pl.when(my == 0)
        def _():
            vhalo[:, pl.ds(0, H)] = jnp.zeros((B, H, HQ, DH), BF)

        @pl.when(my == LAST)
        def _():
            vhalo[:, pl.ds(H, H)] = jnp.zeros((B, H, HQ, DH), BF)

        kc[...] = k_ref[...].astype(BF)
        vc[...] = v_ref[...].astype(BF)

        xv = x_ref[:, :, :].reshape(B * S, 512).astype(BF)
        q = jnp.dot(xv, wq_ref[:, :].astype(BF),
                    preferred_element_type=jnp.float32)
        q = q.reshape(B, S, HQ, DH).astype(BF)

        r_c = lax.broadcasted_iota(jnp.int32, (S, S), 0)
        c_c = lax.broadcasted_iota(jnp.int32, (S, S), 1)
        mask_c = jnp.abs(r_c - c_c) <= H
        r_h = lax.broadcasted_iota(jnp.int32, (H, H), 0)
        c_h = lax.broadcasted_iota(jnp.int32, (H, H), 1)
        mask_l = (c_h >= r_h) & (my > 0)
        mask_r = (c_h <= r_h) & (my < LAST)

        kcv = kc[...]
        vcv = vc[...]

        ctx_c = []
        den = []
        for b in range(B):
            for h in range(HQ):
                s = jax.lax.dot_general(
                    q[b, :, h, :], kcv[b, :, h, :], (((1,), (1,)), ((), ())),
                    preferred_element_type=jnp.float32,
                ) * 0.125
                w = jnp.exp(jnp.where(mask_c, s, -1e9))
                den.append(jnp.sum(w, axis=-1, keepdims=True))
                ctx_c.append(jnp.dot(w.astype(BF), vcv[b, :, h, :],
                                     preferred_element_type=jnp.float32))

        @pl.when(my > 0)
        def _():
            rdma_right_k().wait_recv()
            rdma_right_v().wait_recv()
            rdma_left_k().wait_send()
            rdma_left_v().wait_send()

        @pl.when(my < LAST)
        def _():
            rdma_left_k().wait_recv()
            rdma_left_v().wait_recv()
            rdma_right_k().wait_send()
            rdma_right_v().wait_send()

        khv = khalo[...]
        vhv = vhalo[...]

        for b in range(B):
            ctx_heads = []
            for h in range(HQ):
                i = b * HQ + h
                s_l = jax.lax.dot_general(
                    q[b, 0:H, h, :], khv[b, 0:H, h, :],
                    (((1,), (1,)), ((), ())),
                    preferred_element_type=jnp.float32,
                ) * 0.125
                w_l = jnp.exp(jnp.where(mask_l, s_l, -1e9))
                ctx_l = jnp.dot(w_l.astype(BF), vhv[b, 0:H, h, :],
                                preferred_element_type=jnp.float32)
                d_l = jnp.sum(w_l, axis=-1, keepdims=True)
                s_r = jax.lax.dot_general(
                    q[b, H:2 * H, h, :], khv[b, H:2 * H, h, :],
                    (((1,), (1,)), ((), ())),
                    preferred_element_type=jnp.float32,
                ) * 0.125
                w_r = jnp.exp(jnp.where(mask_r, s_r, -1e9))
                ctx_r = jnp.dot(w_r.astype(BF), vhv[b, H:2 * H, h, :],
                                preferred_element_type=jnp.float32)
                d_r = jnp.sum(w_r, axis=-1, keepdims=True)

                ctx_h = jnp.concatenate([ctx_l, ctx_r], axis=0)
                d_h = jnp.concatenate([d_l, d_r], axis=0)
                ctx_heads.append((ctx_c[i] + ctx_h) / (den[i] + d_h))
            ctx = jnp.concatenate(ctx_heads, axis=-1)
            out_ref[b, :, :] = jnp.dot(
                ctx.astype(BF), wo_ref[:, :].astype(BF),
                preferred_element_type=jnp.float32)

    return pl.pallas_call(
        body,
        out_shape=jax.ShapeDtypeStruct((B, S, 512), jnp.float32),
        in_specs=[pl.BlockSpec(memory_space=pltpu.VMEM)] * 5,
        out_specs=pl.BlockSpec(memory_space=pltpu.VMEM),
        scratch_shapes=[
            pltpu.VMEM((B, 2 * H, HQ, DH), BF),
            pltpu.VMEM((B, 2 * H, HQ, DH), BF),
            pltpu.VMEM((4, B, H, HQ, DH), BF),
            pltpu.VMEM((B, S, HQ, DH), BF),
            pltpu.VMEM((B, S, HQ, DH), BF),
            pltpu.SemaphoreType.DMA((4,)),
            pltpu.SemaphoreType.DMA((4,)),
        ],
        compiler_params=pltpu.CompilerParams(collective_id=0),
    )(x, Wq, K_ext, V_ext, Wo)


# device time: 22085 ns/iter; 1.1395x vs baseline; 1.0193x over previous
import jax
import jax.numpy as jnp
from jax import lax
from jax.experimental import pallas as pl
from jax.experimental.pallas import tpu as pltpu

N_DEV = 32
LAST = N_DEV - 1
B = 2
S = 256
H = 128
HQ = 4
DH = 64
SQ_GLOBAL = N_DEV * S
BF = jnp.bfloat16


def kernel(x, Wq, K_ext, V_ext, Wo):
    def body(x_ref, wq_ref, k_ref, v_ref, wo_ref, out_ref,
             kvhalo, stage, kc, vc, send_sems, recv_sems):
        my = lax.axis_index("i")
        left = my - 1
        right = my + 1

        barrier_sem = pltpu.get_barrier_semaphore()

        @pl.when(my > 0)
        def _():
            pl.semaphore_signal(
                barrier_sem, inc=1,
                device_id=(left,), device_id_type=pl.DeviceIdType.MESH,
            )

        @pl.when(my < LAST)
        def _():
            pl.semaphore_signal(
                barrier_sem, inc=1,
                device_id=(right,), device_id_type=pl.DeviceIdType.MESH,
            )

        stage[0, 0] = k_ref[:, pl.ds(0, H)].astype(BF)
        stage[0, 1] = v_ref[:, pl.ds(0, H)].astype(BF)
        stage[1, 0] = k_ref[:, pl.ds(S - H, H)].astype(BF)
        stage[1, 1] = v_ref[:, pl.ds(S - H, H)].astype(BF)

        @pl.when(my > 0)
        def _():
            pl.semaphore_wait(barrier_sem, 1)

        @pl.when(my < LAST)
        def _():
            pl.semaphore_wait(barrier_sem, 1)

        def rdma_right():
            return pltpu.make_async_remote_copy(
                src_ref=stage.at[1], dst_ref=kvhalo.at[0],
                send_sem=send_sems.at[0], recv_sem=recv_sems.at[0],
                device_id=(right,), device_id_type=pl.DeviceIdType.MESH,
            )

        def rdma_left():
            return pltpu.make_async_remote_copy(
                src_ref=stage.at[0], dst_ref=kvhalo.at[1],
                send_sem=send_sems.at[1], recv_sem=recv_sems.at[1],
                device_id=(left,), device_id_type=pl.DeviceIdType.MESH,
            )

        @pl.when(my < LAST)
        def _():
            rdma_right().start()

        @pl.when(my > 0)
        def _():
            rdma_left().start()

        @pl.when(my == 0)
        def _():
            kvhalo[0, 1] = jnp.zeros((B, H, HQ, DH), BF)

        @pl.when(my == LAST)
        def _():
            kvhalo[1, 1] = jnp.zeros((B, H, HQ, DH), BF)

        kc[...] = k_ref[...].astype(BF)
        vc[...] = v_ref[...].astype(BF)

        xv = x_ref[:, :, :].reshape(B * S, 512).astype(BF)
        q = jnp.dot(xv, wq_ref[:, :].astype(BF),
                    preferred_element_type=jnp.float32)
        q = q.reshape(B, S, HQ, DH).astype(BF)

        r_c = lax.broadcasted_iota(jnp.int32, (S, S), 0)
        c_c = lax.broadcasted_iota(jnp.int32, (S, S), 1)
        mask_c = jnp.abs(r_c - c_c) <= H
        r_h = lax.broadcasted_iota(jnp.int32, (H, H), 0)
        c_h = lax.broadcasted_iota(jnp.int32, (H, H), 1)
        mask_l = (c_h >= r_h) & (my > 0)
        mask_r = (c_h <= r_h) & (my < LAST)

        kcv = kc[...]
        vcv = vc[...]

        ctx_c = []
        den = []
        for b in range(B):
            for h in range(HQ):
                s = jax.lax.dot_general(
                    q[b, :, h, :], kcv[b, :, h, :], (((1,), (1,)), ((), ())),
                    preferred_element_type=jnp.float32,
                ) * 0.125
                w = jnp.exp(jnp.where(mask_c, s, -1e9))
                den.append(jnp.sum(w, axis=-1, keepdims=True))
                ctx_c.append(jnp.dot(w.astype(BF), vcv[b, :, h, :],
                                     preferred_element_type=jnp.float32))

        @pl.when(my > 0)
        def _():
            rdma_right().wait_recv()
            rdma_left().wait_send()

        @pl.when(my < LAST)
        def _():
            rdma_left().wait_recv()
            rdma_right().wait_send()

        khv_l = kvhalo[0, 0]
        vhv_l = kvhalo[0, 1]
        khv_r = kvhalo[1, 0]
        vhv_r = kvhalo[1, 1]

        for b in range(B):
            ctx_heads = []
            for h in range(HQ):
                i = b * HQ + h
                s_l = jax.lax.dot_general(
                    q[b, 0:H, h, :], khv_l[b, :, h, :],
                    (((1,), (1,)), ((), ())),
                    preferred_element_type=jnp.float32,
                ) * 0.125
                w_l = jnp.exp(jnp.where(mask_l, s_l, -1e9))
                ctx_l = jnp.dot(w_l.astype(BF), vhv_l[b, :, h, :],
                                preferred_element_type=jnp.float32)
                d_l = jnp.sum(w_l, axis=-1, keepdims=True)
                s_r = jax.lax.dot_general(
                    q[b, H:2 * H, h, :], khv_r[b, :, h, :],
                    (((1,), (1,)), ((), ())),
                    preferred_element_type=jnp.float32,
                ) * 0.125
                w_r = jnp.exp(jnp.where(mask_r, s_r, -1e9))
                ctx_r = jnp.dot(w_r.astype(BF), vhv_r[b, :, h, :],
                                preferred_element_type=jnp.float32)
                d_r = jnp.sum(w_r, axis=-1, keepdims=True)

                ctx_h = jnp.concatenate([ctx_l, ctx_r], axis=0)
                d_h = jnp.concatenate([d_l, d_r], axis=0)
                ctx_heads.append((ctx_c[i] + ctx_h) / (den[i] + d_h))
            ctx = jnp.concatenate(ctx_heads, axis=-1)
            out_ref[b, :, :] = jnp.dot(
                ctx.astype(BF), wo_ref[:, :].astype(BF),
                preferred_element_type=jnp.float32)

    return pl.pallas_call(
        body,
        out_shape=jax.ShapeDtypeStruct((B, S, 512), jnp.float32),
        in_specs=[pl.BlockSpec(memory_space=pltpu.VMEM)] * 5,
        out_specs=pl.BlockSpec(memory_space=pltpu.VMEM),
        scratch_shapes=[
            pltpu.VMEM((2, 2, B, H, HQ, DH), BF),
            pltpu.VMEM((2, 2, B, H, HQ, DH), BF),
            pltpu.VMEM((B, S, HQ, DH), BF),
            pltpu.VMEM((B, S, HQ, DH), BF),
            pltpu.SemaphoreType.DMA((2,)),
            pltpu.SemaphoreType.DMA((2,)),
        ],
        compiler_params=pltpu.CompilerParams(collective_id=0),
    )(x, Wq, K_ext, V_ext, Wo)
